# baseline (speedup 1.0000x reference)
import functools

import jax
import jax.numpy as jnp
from jax import lax
from jax.experimental import pallas as pl
from jax.experimental.pallas import tpu as pltpu
from jax.experimental.pallas import tpu_sc as plsc

_NB, _NS = 4096, 200
_B = _NB * _NS
_D = 64
_NC, _NSUB = 2, 16
_NW = _NC * _NSUB
_BPW = _B // _NW
_CH = 128
_NCH = _BPW // _CH

_mesh = plsc.VectorSubcoreMesh(core_axis_name="c", subcore_axis_name="s")


@functools.partial(
    pl.kernel,
    mesh=_mesh,
    out_type=jax.ShapeDtypeStruct((_B, _D), jnp.float32),
    scratch_types=[
        pltpu.VMEM((_NCH, _CH), jnp.int32),
        pltpu.VMEM((_CH, _D), jnp.float32),
        pltpu.SemaphoreType.DMA,
    ],
    compiler_params=pltpu.CompilerParams(use_tc_tiling_on_sc=False),
)
def _emb_lookup(ids_hbm, w_hbm, out_hbm, idx_v, rows_v, sem):
    wid = lax.axis_index("s") * _NC + lax.axis_index("c")
    base = wid * _BPW
    pltpu.sync_copy(ids_hbm.at[wid], idx_v)

    def body(g, carry):
        pltpu.async_copy(w_hbm.at[idx_v.at[g]], rows_v, sem).wait()
        pltpu.sync_copy(rows_v, out_hbm.at[pl.ds(base + g * _CH, _CH)])
        return carry

    lax.fori_loop(0, _NCH, body, 0)


def kernel(ids, weight):
    flat_ids = ids.reshape(_NW, _NCH, _CH).astype(jnp.int32)
    out = _emb_lookup(flat_ids, weight)
    return out.reshape(_NB, _NS, _D)


# speedup vs baseline: 1.1144x; 1.1144x over previous
import functools

import jax
import jax.numpy as jnp
from jax import lax
from jax.experimental import pallas as pl
from jax.experimental.pallas import tpu as pltpu
from jax.experimental.pallas import tpu_sc as plsc

_NB, _NS = 4096, 200
_B = _NB * _NS
_D = 64
_NC, _NSUB = 2, 16
_NW = _NC * _NSUB
_BPW = _B // _NW
_CH = 128
_NCH = _BPW // _CH
_NBUF = 8
_LA = 4

_mesh = plsc.VectorSubcoreMesh(core_axis_name="c", subcore_axis_name="s")


@functools.partial(
    pl.kernel,
    mesh=_mesh,
    out_type=jax.ShapeDtypeStruct((_B, _D), jnp.float32),
    scratch_types=[
        pltpu.VMEM((_NCH, _CH), jnp.int32),
        pltpu.VMEM((_NBUF, _CH, _D), jnp.float32),
        [pltpu.SemaphoreType.DMA] * _NBUF,
        [pltpu.SemaphoreType.DMA] * _NBUF,
    ],
    compiler_params=pltpu.CompilerParams(use_tc_tiling_on_sc=False),
)
def _emb_lookup(ids_hbm, w_hbm, out_hbm, idx_v, rows_v, gsems, psems):
    wid = lax.axis_index("s") * _NC + lax.axis_index("c")
    base = wid * _BPW
    pltpu.sync_copy(ids_hbm.at[wid], idx_v)

    def gather(g, b):
        pltpu.async_copy(w_hbm.at[idx_v.at[g]], rows_v.at[b], gsems[b])

    def put(g, b):
        pltpu.async_copy(
            rows_v.at[b], out_hbm.at[pl.ds(base + g * _CH, _CH)], psems[b]
        )

    def wait_gather(b):
        pltpu.make_async_copy(w_hbm.at[idx_v.at[0]], rows_v.at[b], gsems[b]).wait()

    def wait_put(b):
        pltpu.make_async_copy(
            rows_v.at[b], out_hbm.at[pl.ds(base, _CH)], psems[b]
        ).wait()

    for b in range(_LA):
        gather(b, b)

    def outer(i, carry):
        g0 = i * _NBUF
        for b in range(_NBUF):
            g = g0 + b
            wait_gather(b)
            put(g, b)
            b2 = (b + _LA) % _NBUF

            @pl.when(g >= _NBUF - _LA)
            def _():
                wait_put(b2)

            @pl.when(g + _LA < _NCH)
            def _():
                gather(g + _LA, b2)
        return carry

    lax.fori_loop(0, _NCH // _NBUF, outer, 0)

    for g in range(_NCH - _LA, _NCH):
        wait_put(g % _NBUF)


def kernel(ids, weight):
    flat_ids = ids.reshape(_NW, _NCH, _CH).astype(jnp.int32)
    out = _emb_lookup(flat_ids, weight)
    return out.reshape(_NB, _NS, _D)
